# Initial kernel scaffold; baseline (speedup 1.0000x reference)
import jax
import jax.numpy as jnp
from jax.experimental import pallas as pl


def kernel(token_ids, weights):
    raise NotImplementedError("write your pallas kernel here")


# speedup vs baseline: 3.2985x; 3.2985x over previous
import functools

import jax
import jax.numpy as jnp
from jax import lax
from jax.experimental import pallas as pl
from jax.experimental.pallas import tpu as pltpu
from jax.experimental.pallas import tpu_sc as plsc

_VOCAB = 100000
_D = 128
_N = 4096 * 50

_INFO = plsc.get_sparse_core_info()
_NC = _INFO.num_cores
_NS = _INFO.num_subcores
_NW = _NC * _NS
_PER_W = _N // _NW
_CHUNK = 640
_NSTEPS = _PER_W // _CHUNK

_mesh = plsc.VectorSubcoreMesh(core_axis_name="c", subcore_axis_name="s")


@functools.partial(
    pl.kernel,
    mesh=_mesh,
    out_type=jax.ShapeDtypeStruct((_N, _D), jnp.float32),
    scratch_types=[
        pltpu.VMEM((_PER_W,), jnp.int32),
        pltpu.VMEM((_CHUNK, _D), jnp.float32),
        pltpu.SemaphoreType.DMA,
    ],
)
def _gather_all(tok_hbm, w_hbm, out_hbm, idx_v, rows_v, sem):
    wid = lax.axis_index("s") * _NC + lax.axis_index("c")
    base = wid * _PER_W
    pltpu.sync_copy(tok_hbm.at[pl.ds(base, _PER_W)], idx_v)

    def step(g, carry):
        off = g * _CHUNK
        pltpu.async_copy(
            w_hbm.at[idx_v.at[pl.ds(off, _CHUNK)]], rows_v, sem
        ).wait()
        pltpu.sync_copy(rows_v, out_hbm.at[pl.ds(base + off, _CHUNK)])
        return carry

    lax.fori_loop(0, _NSTEPS, step, 0)


def kernel(token_ids, weights):
    flat = token_ids.reshape(-1).astype(jnp.int32)
    out = _gather_all(flat, weights)
    return out.reshape(token_ids.shape + (weights.shape[1],))


# speedup vs baseline: 3.3394x; 1.0124x over previous
import functools

import jax
import jax.numpy as jnp
from jax import lax
from jax.experimental import pallas as pl
from jax.experimental.pallas import tpu as pltpu
from jax.experimental.pallas import tpu_sc as plsc

_VOCAB = 100000
_D = 128
_N = 4096 * 50

_INFO = plsc.get_sparse_core_info()
_NC = _INFO.num_cores
_NS = _INFO.num_subcores
_NW = _NC * _NS
_PER_W = _N // _NW
_NBUF = 4
_LEAD = 2
_CHUNK = 200
_NSTEPS = _PER_W // _CHUNK

_mesh = plsc.VectorSubcoreMesh(core_axis_name="c", subcore_axis_name="s")


@functools.partial(
    pl.kernel,
    mesh=_mesh,
    out_type=jax.ShapeDtypeStruct((_N, _D), jnp.float32),
    scratch_types=[
        pltpu.VMEM((_PER_W,), jnp.int32),
        [pltpu.VMEM((_CHUNK, _D), jnp.float32)] * _NBUF,
        [pltpu.SemaphoreType.DMA] * _NBUF,
        [pltpu.SemaphoreType.DMA] * _NBUF,
    ],
)
def _gather_all(tok_hbm, w_hbm, out_hbm, idx_v, bufs, gsems, ssems):
    wid = lax.axis_index("s") * _NC + lax.axis_index("c")
    base = wid * _PER_W
    pltpu.sync_copy(tok_hbm.at[pl.ds(base, _PER_W)], idx_v)

    def gather(step, b):
        return pltpu.make_async_copy(
            w_hbm.at[idx_v.at[pl.ds(step * _CHUNK, _CHUNK)]], bufs[b], gsems[b]
        )

    def scatter(step, b):
        return pltpu.make_async_copy(
            bufs[b], out_hbm.at[pl.ds(base + step * _CHUNK, _CHUNK)], ssems[b]
        )

    for s in range(_LEAD):
        gather(s, s % _NBUF).start()

    def group(o, carry):
        for b in range(_NBUF):
            s = o * _NBUF + b
            bn = (b + _LEAD) % _NBUF
            @pl.when(s >= _NBUF - _LEAD)
            def _():
                scatter(s - (_NBUF - _LEAD), bn).wait()

            @pl.when(s + _LEAD < _NSTEPS)
            def _():
                gather(s + _LEAD, bn).start()

            gather(s, b).wait()
            scatter(s, b).start()
        return carry

    lax.fori_loop(0, _NSTEPS // _NBUF, group, 0)

    for s in range(_NSTEPS - (_NBUF - _LEAD), _NSTEPS):
        scatter(s, s % _NBUF).wait()


def kernel(token_ids, weights):
    flat = token_ids.reshape(-1).astype(jnp.int32)
    out = _gather_all(flat, weights)
    return out.reshape(token_ids.shape + (weights.shape[1],))


# speedup vs baseline: 5.9840x; 1.7920x over previous
import functools

import jax
import jax.numpy as jnp
from jax import lax
from jax.experimental import pallas as pl
from jax.experimental.pallas import tpu as pltpu
from jax.experimental.pallas import tpu_sc as plsc

_VOCAB = 100000
_D = 128
_B = 4096
_H = 50

_INFO = plsc.get_sparse_core_info()
_NC = _INFO.num_cores
_NS = _INFO.num_subcores
_NW = _NC * _NS
_B_PER_W = _B // _NW
_NBUF = 4
_LEAD = 2
_CHUNK_B = 4
_NSTEPS = _B_PER_W // _CHUNK_B

_mesh = plsc.VectorSubcoreMesh(core_axis_name="c", subcore_axis_name="s")


@functools.partial(
    pl.kernel,
    mesh=_mesh,
    out_type=jax.ShapeDtypeStruct((_B, _H, _D), jnp.float32),
    scratch_types=[
        pltpu.VMEM((_B_PER_W, _H), jnp.int32),
        [pltpu.VMEM((_CHUNK_B, _H, _D), jnp.float32)] * _NBUF,
        [pltpu.SemaphoreType.DMA] * _NBUF,
        [pltpu.SemaphoreType.DMA] * _NBUF,
    ],
)
def _gather_all(tok_hbm, w_hbm, out_hbm, idx_v, bufs, gsems, ssems):
    wid = lax.axis_index("s") * _NC + lax.axis_index("c")
    base = wid * _B_PER_W
    pltpu.sync_copy(tok_hbm.at[pl.ds(base, _B_PER_W)], idx_v)

    def gather_start(step, b):
        for i in range(_CHUNK_B):
            pltpu.make_async_copy(
                w_hbm.at[idx_v.at[step * _CHUNK_B + i]],
                bufs[b].at[i],
                gsems[b],
            ).start()

    def gather_wait(step, b):
        pltpu.make_async_copy(
            out_hbm.at[pl.ds(base + step * _CHUNK_B, _CHUNK_B)],
            bufs[b],
            gsems[b],
        ).wait()

    def scatter(step, b):
        return pltpu.make_async_copy(
            bufs[b],
            out_hbm.at[pl.ds(base + step * _CHUNK_B, _CHUNK_B)],
            ssems[b],
        )

    for s in range(_LEAD):
        gather_start(s, s % _NBUF)

    def group(o, carry):
        for b in range(_NBUF):
            s = o * _NBUF + b
            bn = (b + _LEAD) % _NBUF
            @pl.when(s >= _NBUF - _LEAD)
            def _():
                scatter(s - (_NBUF - _LEAD), bn).wait()

            @pl.when(s + _LEAD < _NSTEPS)
            def _():
                gather_start(s + _LEAD, bn)

            gather_wait(s, b)
            scatter(s, b).start()
        return carry

    lax.fori_loop(0, _NSTEPS // _NBUF, group, 0)

    for s in range(_NSTEPS - (_NBUF - _LEAD), _NSTEPS):
        scatter(s, s % _NBUF).wait()


def kernel(token_ids, weights):
    return _gather_all(token_ids.astype(jnp.int32), weights)
